# baseline (speedup 1.0000x reference)
import functools

import jax
import jax.numpy as jnp
from jax import lax
from jax.experimental import pallas as pl
from jax.experimental.pallas import tpu as pltpu
from jax.experimental.pallas import tpu_sc as plsc

_NS = [40962, 10242, 2562, 642, 162, 42]
_CHS = [3, 32, 64, 128, 256, 512]
_CHP = [16, 32, 64, 128, 256, 512]
_UPS = [(1, 42, 162, 4), (2, 162, 642, 3), (3, 642, 2562, 2),
        (4, 2562, 10242, 1), (5, 10242, 40962, 0)]
_UOC = {1: 256, 2: 128, 3: 64, 4: 32, 5: 3}

_NCORES = 2
_NSUB = 16
_NW = _NCORES * _NSUB

@functools.cache
def _sc_mesh():
    return plsc.VectorSubcoreMesh(core_axis_name="c", subcore_axis_name="s",
                                  num_cores=_NCORES, num_subcores=_NSUB)


def _ceil_to(x, m):
    return -(-x // m) * m


@functools.cache
def _gather_call(C, n_out_pad, group, scale=None, n_valid=None):
    stats = n_valid is not None
    assert n_out_pad % (8 * _NW) == 0 and C % 16 == 0
    bpw = n_out_pad // _NW
    cap = max(8, min(1024, (24576 // (C * group)) // 8 * 8))
    n_chunks = -(-bpw // cap)
    chunk = _ceil_to(-(-bpw // n_chunks), 8)
    sizes = []
    off = 0
    while off < bpw:
        sz = min(chunk, bpw - off)
        sizes.append((off, sz))
        off += sz
    nch = len(sizes)
    U = 8
    while U > 1 and U * group * (C // 16) > 32:
        U //= 2
    scratch = [
        [pltpu.VMEM((group * chunk,), jnp.int32) for _ in range(2)],
        [pltpu.VMEM((group * chunk, C), jnp.float32) for _ in range(2)],
        [pltpu.SemaphoreType.DMA for _ in range(2)],
        [pltpu.SemaphoreType.DMA for _ in range(2)],
    ]
    if group > 1:
        scratch.append([pltpu.VMEM((chunk, C), jnp.float32) for _ in range(2)])
    if stats:
        scratch.append(pltpu.VMEM((16, C), jnp.float32))
    out_type = jax.ShapeDtypeStruct((n_out_pad, C), jnp.float32)
    if stats:
        out_type = (out_type,
                    jax.ShapeDtypeStruct((_NW * 8, C), jnp.float32),
                    jax.ShapeDtypeStruct((_NW * 8, C), jnp.float32))

    @functools.partial(
        pl.kernel,
        out_type=out_type,
        mesh=_sc_mesh(),
        scratch_types=scratch,
        compiler_params=pltpu.CompilerParams(use_tc_tiling_on_sc=False),
    )
    def gk(table_hbm, idx_hbm, *args):
        if stats:
            out_hbm, ps_hbm, pq_hbm = args[0], args[1], args[2]
            args = args[3:]
        else:
            out_hbm = args[0]
            args = args[1:]
        idx_v, rows_v, gsem, wsem = args[0], args[1], args[2], args[3]
        rest = args[4:]
        out_v = rest[0] if group > 1 else rows_v
        st_v = rest[-1] if stats else None
        wid = lax.axis_index("s") * _NCORES + lax.axis_index("c")
        base = wid * bpw

        def fire(t):
            b = t % 2
            off_, sz = sizes[t]
            gsz = group * sz
            pltpu.sync_copy(idx_hbm.at[pl.ds((base + off_) * group, gsz)],
                            idx_v[b].at[pl.ds(0, gsz)])
            handles = []
            s0 = 0
            while s0 < gsz:
                s = min(512, gsz - s0)
                handles.append(pltpu.async_copy(
                    table_hbm.at[idx_v[b].at[pl.ds(s0, s)]],
                    rows_v[b].at[pl.ds(s0, s)], gsem[b]))
                s0 += s
            return handles

        gh = {0: fire(0)}
        wb = {}
        nc16 = C // 16
        zero = jnp.zeros((16,), jnp.float32)
        st_carry = tuple([zero] * (2 * nc16))
        for t in range(nch):
            b = t % 2
            if t + 1 < nch:
                if t + 1 >= 2 and wb.get((t + 1) % 2) is not None:
                    wb[(t + 1) % 2].wait()
                    wb[(t + 1) % 2] = None
                gh[t + 1] = fire(t + 1)
            for h in gh.pop(t):
                h.wait()
            off_, sz = sizes[t]
            if group > 1:
                inv = jnp.float32(scale if scale is not None else 1.0 / group)

                def body_j(j, carry):
                    carry = list(carry)
                    j0 = j * U
                    for u in range(U):
                        for c0 in range(nc16):
                            acc = rows_v[b][(j0 + u) * group,
                                            pl.ds(c0 * 16, 16)]
                            for k in range(1, group):
                                acc = acc + rows_v[b][(j0 + u) * group + k,
                                                      pl.ds(c0 * 16, 16)]
                            acc = acc * inv
                            out_v[b][j0 + u, pl.ds(c0 * 16, 16)] = acc
                            if stats:
                                accm = jnp.where(
                                    base + off_ + j0 + u < n_valid, acc, 0.0)
                                carry[c0] = carry[c0] + accm
                                carry[nc16 + c0] = (carry[nc16 + c0]
                                                    + accm * accm)
                    return tuple(carry)

                st_carry = lax.fori_loop(0, sz // U, body_j, st_carry)
            wb[b] = pltpu.async_copy(out_v[b].at[pl.ds(0, sz)],
                                     out_hbm.at[pl.ds(base + off_, sz)],
                                     wsem[b])
        if stats:
            for r in range(2):
                for c0 in range(nc16):
                    st_v[8 * r, pl.ds(c0 * 16, 16)] = st_carry[r * nc16 + c0]
                for rr in range(1, 8):
                    for c0 in range(nc16):
                        st_v[8 * r + rr, pl.ds(c0 * 16, 16)] = zero
            pltpu.sync_copy(st_v.at[pl.ds(0, 8)], ps_hbm.at[pl.ds(wid * 8, 8)])
            pltpu.sync_copy(st_v.at[pl.ds(8, 8)], pq_hbm.at[pl.ds(wid * 8, 8)])
        for b in (0, 1):
            if wb.get(b) is not None:
                wb[b].wait()

    return gk


def _gather(table, idx):
    M = idx.shape[0]
    C = table.shape[1]
    M_pad = _ceil_to(M, 8 * _NW)
    idxp = jnp.pad(idx, (0, M_pad - M)) if M_pad != M else idx
    out = _gather_call(C, M_pad, 1)(table, idxp)
    return out[:M] if M_pad != M else out


def _gather_mean(table, idx, group):
    n_out = idx.shape[0] // group
    C = table.shape[1]
    n_out_pad = _ceil_to(n_out, 8 * _NW)
    idxp = jnp.pad(idx, (0, group * (n_out_pad - n_out)))
    return _gather_call(C, n_out_pad, group)(table, idxp)


def _gather_sum_stats(table, idx, group, n_valid):
    n_out = idx.shape[0] // group
    C = table.shape[1]
    n_out_pad = _ceil_to(n_out, 8 * _NW)
    idxp = jnp.pad(idx, (0, group * (n_out_pad - n_out)))
    return _gather_call(C, n_out_pad, group, 1.0, n_valid)(table, idxp)


def _dot(a, b, dims):
    return lax.dot_general(a, b, (dims, ((), ())),
                           precision=lax.Precision.HIGHEST,
                           preferred_element_type=jnp.float32)


def _conv_bn_body(sigmoid, m_ref, w_ref, b_ref, g_ref, be_ref, o_ref):
    h2 = _dot(m_ref[:], w_ref[:], ((1,), (1,))) + b_ref[:]
    mu = jnp.mean(h2, axis=0, keepdims=True)
    var = jnp.mean((h2 - mu) ** 2, axis=0, keepdims=True)
    h2 = (h2 - mu) / jnp.sqrt(var + 1e-5) * g_ref[:] + be_ref[:]
    h2 = jnp.where(h2 >= 0, h2, 0.2 * h2)
    if sigmoid:
        h2 = 1.0 / (1.0 + jnp.exp(-h2))
    o_ref[:] = h2


_CONV_BLOCK = 1024


def _mm_stats_body(n, bn, m_ref, w_ref, b_ref, h2_ref, st_ref):
    i = pl.program_id(0)
    h2 = _dot(m_ref[:], w_ref[:], ((1,), (1,))) + b_ref[:]
    h2_ref[:] = h2
    rows = lax.broadcasted_iota(jnp.int32, (bn, 1), 0)
    valid = rows < (n - i * bn)
    h2m = jnp.where(valid, h2, 0.0)
    s = jnp.sum(h2m, axis=0, keepdims=True)
    s2 = jnp.sum(h2m * h2m, axis=0, keepdims=True)

    @pl.when(i == 0)
    def _():
        st_ref[:] = jnp.zeros_like(st_ref)

    st_ref[0:1, :] += s
    st_ref[1:2, :] += s2


def _bn_act_body(n, sigmoid, h2_ref, st_ref, g_ref, be_ref, o_ref):
    mu = st_ref[0:1, :] / n
    var = st_ref[1:2, :] / n - mu * mu
    h2 = (h2_ref[:] - mu) / jnp.sqrt(var + 1e-5) * g_ref[:] + be_ref[:]
    h2 = jnp.where(h2 >= 0, h2, 0.2 * h2)
    if sigmoid:
        h2 = 1.0 / (1.0 + jnp.exp(-h2))
    o_ref[:] = h2


def _conv_bn(mat, W, b, g, be, sigmoid=False):
    n, kdim = mat.shape
    ocp = W.shape[0]
    b2, g2, be2 = b.reshape(1, -1), g.reshape(1, -1), be.reshape(1, -1)
    if n <= 2562:
        return pl.pallas_call(
            functools.partial(_conv_bn_body, sigmoid),
            out_shape=jax.ShapeDtypeStruct((n, ocp), jnp.float32),
        )(mat, W, b2, g2, be2)
    bn = _CONV_BLOCK
    nb = -(-n // bn)
    h2, st = pl.pallas_call(
        functools.partial(_mm_stats_body, n, bn),
        grid=(nb,),
        in_specs=[
            pl.BlockSpec((bn, kdim), lambda i: (i, 0)),
            pl.BlockSpec((ocp, kdim), lambda i: (0, 0)),
            pl.BlockSpec((1, ocp), lambda i: (0, 0)),
        ],
        out_specs=[
            pl.BlockSpec((bn, ocp), lambda i: (i, 0)),
            pl.BlockSpec((8, ocp), lambda i: (0, 0)),
        ],
        out_shape=[
            jax.ShapeDtypeStruct((n, ocp), jnp.float32),
            jax.ShapeDtypeStruct((8, ocp), jnp.float32),
        ],
    )(mat, W, b2)
    return pl.pallas_call(
        functools.partial(_bn_act_body, float(n), sigmoid),
        grid=(nb,),
        in_specs=[
            pl.BlockSpec((bn, ocp), lambda i: (i, 0)),
            pl.BlockSpec((8, ocp), lambda i: (0, 0)),
            pl.BlockSpec((1, ocp), lambda i: (0, 0)),
            pl.BlockSpec((1, ocp), lambda i: (0, 0)),
        ],
        out_specs=pl.BlockSpec((bn, ocp), lambda i: (i, 0)),
        out_shape=jax.ShapeDtypeStruct((n, ocp), jnp.float32),
    )(h2, st, g2, be2)


def _zmat_body(h_ref, w_ref, o_ref):
    o_ref[:] = _dot(h_ref[:], w_ref[:], ((1,), (0,)))


def _zmat(h, Wcat):
    nh, icp = h.shape
    w = Wcat.shape[1]
    if nh <= 8192:
        return pl.pallas_call(
            _zmat_body,
            out_shape=jax.ShapeDtypeStruct((nh, w), jnp.float32),
        )(h, Wcat)
    bn = 2048
    nb = -(-nh // bn)
    return pl.pallas_call(
        _zmat_body,
        grid=(nb,),
        in_specs=[
            pl.BlockSpec((bn, icp), lambda i: (i, 0)),
            pl.BlockSpec((icp, w), lambda i: (0, 0)),
        ],
        out_specs=pl.BlockSpec((bn, w), lambda i: (i, 0)),
        out_shape=jax.ShapeDtypeStruct((nh, w), jnp.float32),
    )(h, Wcat)


def _bn_act_stats_body(n, f, sigmoid, h2_ref, ps_ref, pq_ref, g_ref, be_ref,
                       o_ref):
    mu = jnp.sum(ps_ref[:], axis=0, keepdims=True) / n
    var = jnp.sum(pq_ref[:], axis=0, keepdims=True) / n - mu * mu
    muf = jnp.concatenate([mu] * f, axis=1)
    varf = jnp.concatenate([var] * f, axis=1)
    gf = jnp.concatenate([g_ref[:]] * f, axis=1)
    bef = jnp.concatenate([be_ref[:]] * f, axis=1)
    h2 = (h2_ref[:] - muf) / jnp.sqrt(varf + 1e-5) * gf + bef
    h2 = jnp.where(h2 >= 0, h2, 0.2 * h2)
    if sigmoid:
        h2 = 1.0 / (1.0 + jnp.exp(-h2))
    o_ref[:] = h2


def _bn_act_stats(h2, ps, pq, g, be, n, f, sigmoid):
    n_pad, ocp = h2.shape
    hf = h2.reshape(n_pad // f, f * ocp)
    out = pl.pallas_call(
        functools.partial(_bn_act_stats_body, float(n), f, sigmoid),
        out_shape=jax.ShapeDtypeStruct(hf.shape, jnp.float32),
    )(hf, ps, pq, g.reshape(1, -1), be.reshape(1, -1))
    return out.reshape(n_pad, ocp)


def _upconv_body(h_ref, w_ref, b_ref, o_ref):
    o_ref[:] = _dot(h_ref[:], w_ref[:], ((1,), (0,))) + b_ref[:]


def _upconv(h, WuT, bu, f):
    nh, ic = h.shape
    hf = h.reshape(nh // f, f * ic)
    Wb = jnp.kron(jnp.eye(f, dtype=jnp.float32), WuT) if f > 1 else WuT
    bb = jnp.tile(bu, f)
    return pl.pallas_call(
        _upconv_body,
        out_shape=jax.ShapeDtypeStruct((nh // f, f * WuT.shape[1]),
                                       jnp.float32),
    )(hf, Wb, bb.reshape(1, -1))


def _pad_cols(a, cp):
    return a if a.shape[1] == cp else jnp.pad(a, ((0, 0), (0, cp - a.shape[1])))


def _pad_conv_params(params, name, ic, icp, oc, ocp):
    W = params[name + '_W']
    W = W.reshape(oc, 7, ic)
    W = jnp.pad(W, ((0, ocp - oc), (0, 0), (0, icp - ic))).reshape(ocp, 7 * icp)
    b = jnp.pad(params[name + '_b'], (0, ocp - oc))
    g = jnp.pad(params[name + '_g'], (0, ocp - oc))
    be = jnp.pad(params[name + '_be'], (0, ocp - oc))
    return W, b, g, be


def _conv_layer(h, no, params, name, ic, icp, oc, ocp, sigmoid=False):
    n = no.shape[0] // 7
    W, b, g, be = _pad_conv_params(params, name, ic, icp, oc, ocp)
    if n >= 2562:
        f = max(1, 128 // icp)
        fo = max(1, 128 // ocp)
        nh = h.shape[0]
        hf = h.reshape(nh // f, f * icp)
        Wcat = W.reshape(ocp, 7, icp).transpose(2, 1, 0).reshape(icp, 7 * ocp)
        Wbig = (jnp.kron(jnp.eye(f, dtype=jnp.float32), Wcat)
                if f > 1 else Wcat)
        Zf = _zmat(hf, Wbig)
        ztab = Zf.reshape(nh * 7, ocp)
        no2 = (no.reshape(n, 7) * 7
               + jnp.arange(7, dtype=jnp.int32)).reshape(-1)
        h2, ps, pq = _gather_sum_stats(ztab, no2, 7, n)
        return _bn_act_stats(h2, ps, pq, g, be, n, fo, sigmoid)
    gth = _gather(h, no)
    mat = gth.reshape(n, 7 * icp)
    return _conv_bn(mat, W, b, g, be, sigmoid=sigmoid)


def kernel(x, params, idx):
    h = _pad_cols(x, _CHP[0])
    for i in range(1, 6):
        n = _NS[i]
        hp = _gather_mean(h, idx['no%d' % (i - 1)][: n * 7], 7)
        h = _conv_layer(hp, idx['no%d' % i], params, 'd%dc1' % i,
                        _CHS[i - 1], _CHP[i - 1], _CHS[i], _CHP[i])
        h = _conv_layer(h, idx['no%d' % i], params, 'd%dc2' % i,
                        _CHS[i], _CHP[i], _CHS[i], _CHP[i])
    for (i, nf, nt, lvl) in _UPS:
        ic = _CHS[lvl + 1]
        oc = _UOC[i]
        ocp = 16 if oc == 3 else oc
        Wu = params['u%d_W' % i].reshape(7, oc, ic)
        Wu = jnp.pad(Wu, ((0, 0), (0, ocp - oc), (0, 0))).reshape(7 * ocp, ic)
        bu = jnp.pad(params['u%d_b' % i].reshape(7, oc),
                     ((0, 0), (0, ocp - oc))).reshape(7 * ocp)
        fu = max(1, 128 // ic) if h.shape[0] % max(1, 128 // ic) == 0 else 1
        yf = _upconv(h, Wu.T, bu, fu)
        ytab = yf.reshape(h.shape[0] * 7, ocp)
        comb = jnp.concatenate(
            [jnp.repeat(idx['top%d' % nt], 2), idx['down%d' % nt]])
        h = _gather_mean(ytab, comb, 2)
        h = _conv_layer(h, idx['no%d' % lvl], params, 'u%dc1' % i,
                        oc, ocp, oc, ocp)
        h = _conv_layer(h, idx['no%d' % lvl], params, 'u%dc2' % i,
                        oc, ocp, oc, ocp, sigmoid=(i == 5))
    return h[:_NS[0], :3]


# speedup vs baseline: 1.0115x; 1.0115x over previous
import functools

import jax
import jax.numpy as jnp
from jax import lax
from jax.experimental import pallas as pl
from jax.experimental.pallas import tpu as pltpu
from jax.experimental.pallas import tpu_sc as plsc

_NS = [40962, 10242, 2562, 642, 162, 42]
_CHS = [3, 32, 64, 128, 256, 512]
_CHP = [16, 32, 64, 128, 256, 512]
_UPS = [(1, 42, 162, 4), (2, 162, 642, 3), (3, 642, 2562, 2),
        (4, 2562, 10242, 1), (5, 10242, 40962, 0)]
_UOC = {1: 256, 2: 128, 3: 64, 4: 32, 5: 3}

_NCORES = 2
_NSUB = 16
_NW = _NCORES * _NSUB

@functools.cache
def _sc_mesh():
    return plsc.VectorSubcoreMesh(core_axis_name="c", subcore_axis_name="s",
                                  num_cores=_NCORES, num_subcores=_NSUB)


def _ceil_to(x, m):
    return -(-x // m) * m


@functools.cache
def _gather_call(C, n_out_pad, group, scale=None, n_valid=None):
    stats = n_valid is not None
    assert n_out_pad % (8 * _NW) == 0 and C % 16 == 0
    bpw = n_out_pad // _NW
    cap = max(8, min(1024, (24576 // (C * group)) // 8 * 8))
    n_chunks = -(-bpw // cap)
    chunk = _ceil_to(-(-bpw // n_chunks), 8)
    sizes = []
    off = 0
    while off < bpw:
        sz = min(chunk, bpw - off)
        sizes.append((off, sz))
        off += sz
    nch = len(sizes)
    U = 8
    while U > 1 and U * group * (C // 16) > 32:
        U //= 2
    scratch = [
        [pltpu.VMEM((group * chunk,), jnp.int32) for _ in range(2)],
        [pltpu.VMEM((group * chunk, C), jnp.float32) for _ in range(2)],
        [pltpu.SemaphoreType.DMA for _ in range(2)],
        [pltpu.SemaphoreType.DMA for _ in range(2)],
    ]
    if group > 1:
        scratch.append([pltpu.VMEM((chunk, C), jnp.float32) for _ in range(2)])
    if stats:
        scratch.append(pltpu.VMEM((16, C), jnp.float32))
    out_type = jax.ShapeDtypeStruct((n_out_pad, C), jnp.float32)
    if stats:
        out_type = (out_type,
                    jax.ShapeDtypeStruct((_NW * 8, C), jnp.float32),
                    jax.ShapeDtypeStruct((_NW * 8, C), jnp.float32))

    @functools.partial(
        pl.kernel,
        out_type=out_type,
        mesh=_sc_mesh(),
        scratch_types=scratch,
        compiler_params=pltpu.CompilerParams(use_tc_tiling_on_sc=False),
    )
    def gk(table_hbm, idx_hbm, *args):
        if stats:
            out_hbm, ps_hbm, pq_hbm = args[0], args[1], args[2]
            args = args[3:]
        else:
            out_hbm = args[0]
            args = args[1:]
        idx_v, rows_v, gsem, wsem = args[0], args[1], args[2], args[3]
        rest = args[4:]
        out_v = rest[0] if group > 1 else rows_v
        st_v = rest[-1] if stats else None
        wid = lax.axis_index("s") * _NCORES + lax.axis_index("c")
        base = wid * bpw

        def fire(t):
            b = t % 2
            off_, sz = sizes[t]
            gsz = group * sz
            pltpu.sync_copy(idx_hbm.at[pl.ds((base + off_) * group, gsz)],
                            idx_v[b].at[pl.ds(0, gsz)])
            handles = []
            s0 = 0
            while s0 < gsz:
                s = min(512, gsz - s0)
                handles.append(pltpu.async_copy(
                    table_hbm.at[idx_v[b].at[pl.ds(s0, s)]],
                    rows_v[b].at[pl.ds(s0, s)], gsem[b]))
                s0 += s
            return handles

        gh = {0: fire(0)}
        wb = {}
        nc16 = C // 16
        zero = jnp.zeros((16,), jnp.float32)
        st_carry = tuple([zero] * (2 * nc16))
        for t in range(nch):
            b = t % 2
            if t + 1 < nch:
                if t + 1 >= 2 and wb.get((t + 1) % 2) is not None:
                    wb[(t + 1) % 2].wait()
                    wb[(t + 1) % 2] = None
                gh[t + 1] = fire(t + 1)
            for h in gh.pop(t):
                h.wait()
            off_, sz = sizes[t]
            if group > 1:
                inv = jnp.float32(scale if scale is not None else 1.0 / group)

                def body_j(j, carry):
                    carry = list(carry)
                    j0 = j * U
                    for u in range(U):
                        for c0 in range(nc16):
                            acc = rows_v[b][(j0 + u) * group,
                                            pl.ds(c0 * 16, 16)]
                            for k in range(1, group):
                                acc = acc + rows_v[b][(j0 + u) * group + k,
                                                      pl.ds(c0 * 16, 16)]
                            acc = acc * inv
                            out_v[b][j0 + u, pl.ds(c0 * 16, 16)] = acc
                            if stats:
                                accm = jnp.where(
                                    base + off_ + j0 + u < n_valid, acc, 0.0)
                                carry[c0] = carry[c0] + accm
                                carry[nc16 + c0] = (carry[nc16 + c0]
                                                    + accm * accm)
                    return tuple(carry)

                st_carry = lax.fori_loop(0, sz // U, body_j, st_carry)
            wb[b] = pltpu.async_copy(out_v[b].at[pl.ds(0, sz)],
                                     out_hbm.at[pl.ds(base + off_, sz)],
                                     wsem[b])
        if stats:
            for r in range(2):
                for c0 in range(nc16):
                    st_v[8 * r, pl.ds(c0 * 16, 16)] = st_carry[r * nc16 + c0]
                for rr in range(1, 8):
                    for c0 in range(nc16):
                        st_v[8 * r + rr, pl.ds(c0 * 16, 16)] = zero
            pltpu.sync_copy(st_v.at[pl.ds(0, 8)], ps_hbm.at[pl.ds(wid * 8, 8)])
            pltpu.sync_copy(st_v.at[pl.ds(8, 8)], pq_hbm.at[pl.ds(wid * 8, 8)])
        for b in (0, 1):
            if wb.get(b) is not None:
                wb[b].wait()

    return gk


def _gather(table, idx):
    M = idx.shape[0]
    C = table.shape[1]
    M_pad = _ceil_to(M, 8 * _NW)
    idxp = jnp.pad(idx, (0, M_pad - M)) if M_pad != M else idx
    out = _gather_call(C, M_pad, 1)(table, idxp)
    return out[:M] if M_pad != M else out


def _gather_mean(table, idx, group):
    n_out = idx.shape[0] // group
    C = table.shape[1]
    n_out_pad = _ceil_to(n_out, 8 * _NW)
    idxp = jnp.pad(idx, (0, group * (n_out_pad - n_out)))
    return _gather_call(C, n_out_pad, group)(table, idxp)


def _gather_sum_stats(table, idx, group, n_valid):
    n_out = idx.shape[0] // group
    C = table.shape[1]
    n_out_pad = _ceil_to(n_out, 8 * _NW)
    idxp = jnp.pad(idx, (0, group * (n_out_pad - n_out)))
    return _gather_call(C, n_out_pad, group, 1.0, n_valid)(table, idxp)


def _dot(a, b, dims):
    return lax.dot_general(a, b, (dims, ((), ())),
                           precision=lax.Precision.HIGHEST,
                           preferred_element_type=jnp.float32)


def _conv_bn_body(sigmoid, m_ref, w_ref, b_ref, g_ref, be_ref, o_ref):
    h2 = _dot(m_ref[:], w_ref[:], ((1,), (1,))) + b_ref[:]
    mu = jnp.mean(h2, axis=0, keepdims=True)
    var = jnp.mean((h2 - mu) ** 2, axis=0, keepdims=True)
    h2 = (h2 - mu) / jnp.sqrt(var + 1e-5) * g_ref[:] + be_ref[:]
    h2 = jnp.where(h2 >= 0, h2, 0.2 * h2)
    if sigmoid:
        h2 = 1.0 / (1.0 + jnp.exp(-h2))
    o_ref[:] = h2


_CONV_BLOCK = 1024


def _mm_stats_body(n, bn, m_ref, w_ref, b_ref, h2_ref, st_ref):
    i = pl.program_id(0)
    h2 = _dot(m_ref[:], w_ref[:], ((1,), (1,))) + b_ref[:]
    h2_ref[:] = h2
    rows = lax.broadcasted_iota(jnp.int32, (bn, 1), 0)
    valid = rows < (n - i * bn)
    h2m = jnp.where(valid, h2, 0.0)
    s = jnp.sum(h2m, axis=0, keepdims=True)
    s2 = jnp.sum(h2m * h2m, axis=0, keepdims=True)

    @pl.when(i == 0)
    def _():
        st_ref[:] = jnp.zeros_like(st_ref)

    st_ref[0:1, :] += s
    st_ref[1:2, :] += s2


def _bn_act_body(n, sigmoid, h2_ref, st_ref, g_ref, be_ref, o_ref):
    mu = st_ref[0:1, :] / n
    var = st_ref[1:2, :] / n - mu * mu
    h2 = (h2_ref[:] - mu) / jnp.sqrt(var + 1e-5) * g_ref[:] + be_ref[:]
    h2 = jnp.where(h2 >= 0, h2, 0.2 * h2)
    if sigmoid:
        h2 = 1.0 / (1.0 + jnp.exp(-h2))
    o_ref[:] = h2


def _conv_bn(mat, W, b, g, be, sigmoid=False):
    n, kdim = mat.shape
    ocp = W.shape[0]
    b2, g2, be2 = b.reshape(1, -1), g.reshape(1, -1), be.reshape(1, -1)
    if n <= 2562:
        return pl.pallas_call(
            functools.partial(_conv_bn_body, sigmoid),
            out_shape=jax.ShapeDtypeStruct((n, ocp), jnp.float32),
        )(mat, W, b2, g2, be2)
    bn = _CONV_BLOCK
    nb = -(-n // bn)
    h2, st = pl.pallas_call(
        functools.partial(_mm_stats_body, n, bn),
        grid=(nb,),
        in_specs=[
            pl.BlockSpec((bn, kdim), lambda i: (i, 0)),
            pl.BlockSpec((ocp, kdim), lambda i: (0, 0)),
            pl.BlockSpec((1, ocp), lambda i: (0, 0)),
        ],
        out_specs=[
            pl.BlockSpec((bn, ocp), lambda i: (i, 0)),
            pl.BlockSpec((8, ocp), lambda i: (0, 0)),
        ],
        out_shape=[
            jax.ShapeDtypeStruct((n, ocp), jnp.float32),
            jax.ShapeDtypeStruct((8, ocp), jnp.float32),
        ],
    )(mat, W, b2)
    return pl.pallas_call(
        functools.partial(_bn_act_body, float(n), sigmoid),
        grid=(nb,),
        in_specs=[
            pl.BlockSpec((bn, ocp), lambda i: (i, 0)),
            pl.BlockSpec((8, ocp), lambda i: (0, 0)),
            pl.BlockSpec((1, ocp), lambda i: (0, 0)),
            pl.BlockSpec((1, ocp), lambda i: (0, 0)),
        ],
        out_specs=pl.BlockSpec((bn, ocp), lambda i: (i, 0)),
        out_shape=jax.ShapeDtypeStruct((n, ocp), jnp.float32),
    )(h2, st, g2, be2)


def _zmat_body(h_ref, w_ref, o_ref):
    o_ref[:] = _dot(h_ref[:], w_ref[:], ((1,), (0,)))


def _zmat(h, Wcat):
    nh, icp = h.shape
    w = Wcat.shape[1]
    if nh <= 8192:
        return pl.pallas_call(
            _zmat_body,
            out_shape=jax.ShapeDtypeStruct((nh, w), jnp.float32),
        )(h, Wcat)
    bn = 2048
    nb = -(-nh // bn)
    return pl.pallas_call(
        _zmat_body,
        grid=(nb,),
        in_specs=[
            pl.BlockSpec((bn, icp), lambda i: (i, 0)),
            pl.BlockSpec((icp, w), lambda i: (0, 0)),
        ],
        out_specs=pl.BlockSpec((bn, w), lambda i: (i, 0)),
        out_shape=jax.ShapeDtypeStruct((nh, w), jnp.float32),
    )(h, Wcat)


def _bn_act_stats_body(n, f, sigmoid, h2_ref, ps_ref, pq_ref, g_ref, be_ref,
                       o_ref):
    mu = jnp.sum(ps_ref[:], axis=0, keepdims=True) / n
    var = jnp.sum(pq_ref[:], axis=0, keepdims=True) / n - mu * mu
    muf = jnp.concatenate([mu] * f, axis=1)
    varf = jnp.concatenate([var] * f, axis=1)
    gf = jnp.concatenate([g_ref[:]] * f, axis=1)
    bef = jnp.concatenate([be_ref[:]] * f, axis=1)
    h2 = (h2_ref[:] - muf) / jnp.sqrt(varf + 1e-5) * gf + bef
    h2 = jnp.where(h2 >= 0, h2, 0.2 * h2)
    if sigmoid:
        h2 = 1.0 / (1.0 + jnp.exp(-h2))
    o_ref[:] = h2


def _bn_act_stats(h2, ps, pq, g, be, n, f, sigmoid):
    n_pad, ocp = h2.shape
    hf = h2.reshape(n_pad // f, f * ocp)
    out = pl.pallas_call(
        functools.partial(_bn_act_stats_body, float(n), f, sigmoid),
        out_shape=jax.ShapeDtypeStruct(hf.shape, jnp.float32),
    )(hf, ps, pq, g.reshape(1, -1), be.reshape(1, -1))
    return out.reshape(n_pad, ocp)


def _upconv_body(h_ref, w_ref, b_ref, o_ref):
    o_ref[:] = _dot(h_ref[:], w_ref[:], ((1,), (0,))) + b_ref[:]


def _upconv(h, WuT, bu, f):
    nh, ic = h.shape
    hf = h.reshape(nh // f, f * ic)
    Wb = jnp.kron(jnp.eye(f, dtype=jnp.float32), WuT) if f > 1 else WuT
    bb = jnp.tile(bu, f)
    return pl.pallas_call(
        _upconv_body,
        out_shape=jax.ShapeDtypeStruct((nh // f, f * WuT.shape[1]),
                                       jnp.float32),
    )(hf, Wb, bb.reshape(1, -1))


def _pad_cols(a, cp):
    return a if a.shape[1] == cp else jnp.pad(a, ((0, 0), (0, cp - a.shape[1])))


def _pad_conv_params(params, name, ic, icp, oc, ocp):
    W = params[name + '_W']
    W = W.reshape(oc, 7, ic)
    W = jnp.pad(W, ((0, ocp - oc), (0, 0), (0, icp - ic))).reshape(ocp, 7 * icp)
    b = jnp.pad(params[name + '_b'], (0, ocp - oc))
    g = jnp.pad(params[name + '_g'], (0, ocp - oc))
    be = jnp.pad(params[name + '_be'], (0, ocp - oc))
    return W, b, g, be


def _bn_zmat_body(n, f, h2_ref, ps_ref, pq_ref, g_ref, be_ref, w_ref, o_ref):
    mu = jnp.sum(ps_ref[:], axis=0, keepdims=True) / n
    var = jnp.sum(pq_ref[:], axis=0, keepdims=True) / n - mu * mu
    muf = jnp.concatenate([mu] * f, axis=1)
    varf = jnp.concatenate([var] * f, axis=1)
    gf = jnp.concatenate([g_ref[:]] * f, axis=1)
    bef = jnp.concatenate([be_ref[:]] * f, axis=1)
    h2 = (h2_ref[:] - muf) / jnp.sqrt(varf + 1e-5) * gf + bef
    h2 = jnp.where(h2 >= 0, h2, 0.2 * h2)
    o_ref[:] = _dot(h2, w_ref[:], ((1,), (0,)))


def _bn_zmat(h2, ps, pq, g, be, n, f, Wbig):
    n_pad, ocp = h2.shape
    hf = h2.reshape(n_pad // f, f * ocp)
    return pl.pallas_call(
        functools.partial(_bn_zmat_body, float(n), f),
        out_shape=jax.ShapeDtypeStruct((n_pad // f, Wbig.shape[1]),
                                       jnp.float32),
    )(hf, ps, pq, g.reshape(1, -1), be.reshape(1, -1), Wbig)


def _conv_layer(h, no, params, name, ic, icp, oc, ocp, sigmoid=False):
    n = no.shape[0] // 7
    W, b, g, be = _pad_conv_params(params, name, ic, icp, oc, ocp)
    if n >= 2562:
        f = max(1, 128 // icp)
        fo = max(1, 128 // ocp)
        nh = h.shape[0]
        hf = h.reshape(nh // f, f * icp)
        Wcat = W.reshape(ocp, 7, icp).transpose(2, 1, 0).reshape(icp, 7 * ocp)
        Wbig = (jnp.kron(jnp.eye(f, dtype=jnp.float32), Wcat)
                if f > 1 else Wcat)
        Zf = _zmat(hf, Wbig)
        ztab = Zf.reshape(nh * 7, ocp)
        no2 = (no.reshape(n, 7) * 7
               + jnp.arange(7, dtype=jnp.int32)).reshape(-1)
        h2, ps, pq = _gather_sum_stats(ztab, no2, 7, n)
        return _bn_act_stats(h2, ps, pq, g, be, n, fo, sigmoid)
    gth = _gather(h, no)
    mat = gth.reshape(n, 7 * icp)
    return _conv_bn(mat, W, b, g, be, sigmoid=sigmoid)


def _wcat(W, icp, ocp, f):
    Wcat = W.reshape(ocp, 7, icp).transpose(2, 1, 0).reshape(icp, 7 * ocp)
    return jnp.kron(jnp.eye(f, dtype=jnp.float32), Wcat) if f > 1 else Wcat


def _conv_pair_big(h, no, params, name1, name2, ic, icp, oc, ocp, sigmoid2):
    n = no.shape[0] // 7
    W1, _, g1, be1 = _pad_conv_params(params, name1, ic, icp, oc, ocp)
    W2, _, g2, be2 = _pad_conv_params(params, name2, oc, ocp, oc, ocp)
    f1 = max(1, 128 // icp)
    fo = max(1, 128 // ocp)
    nh = h.shape[0]
    no2 = (no.reshape(n, 7) * 7 + jnp.arange(7, dtype=jnp.int32)).reshape(-1)
    Zf = _zmat(h.reshape(nh // f1, f1 * icp), _wcat(W1, icp, ocp, f1))
    h2a, psa, pqa = _gather_sum_stats(Zf.reshape(nh * 7, ocp), no2, 7, n)
    npd = h2a.shape[0]
    Zf2 = _bn_zmat(h2a, psa, pqa, g1, be1, n, fo, _wcat(W2, ocp, ocp, fo))
    h2b, psb, pqb = _gather_sum_stats(Zf2.reshape(npd * 7, ocp), no2, 7, n)
    return _bn_act_stats(h2b, psb, pqb, g2, be2, n, fo, sigmoid2)


def kernel(x, params, idx):
    h = _pad_cols(x, _CHP[0])
    for i in range(1, 6):
        n = _NS[i]
        hp = _gather_mean(h, idx['no%d' % (i - 1)][: n * 7], 7)
        if n >= 2562:
            h = _conv_pair_big(hp, idx['no%d' % i], params, 'd%dc1' % i,
                               'd%dc2' % i, _CHS[i - 1], _CHP[i - 1],
                               _CHS[i], _CHP[i], False)
        else:
            h = _conv_layer(hp, idx['no%d' % i], params, 'd%dc1' % i,
                            _CHS[i - 1], _CHP[i - 1], _CHS[i], _CHP[i])
            h = _conv_layer(h, idx['no%d' % i], params, 'd%dc2' % i,
                            _CHS[i], _CHP[i], _CHS[i], _CHP[i])
    for (i, nf, nt, lvl) in _UPS:
        ic = _CHS[lvl + 1]
        oc = _UOC[i]
        ocp = 16 if oc == 3 else oc
        Wu = params['u%d_W' % i].reshape(7, oc, ic)
        Wu = jnp.pad(Wu, ((0, 0), (0, ocp - oc), (0, 0))).reshape(7 * ocp, ic)
        bu = jnp.pad(params['u%d_b' % i].reshape(7, oc),
                     ((0, 0), (0, ocp - oc))).reshape(7 * ocp)
        fu = max(1, 128 // ic) if h.shape[0] % max(1, 128 // ic) == 0 else 1
        yf = _upconv(h, Wu.T, bu, fu)
        ytab = yf.reshape(h.shape[0] * 7, ocp)
        comb = jnp.concatenate(
            [jnp.repeat(idx['top%d' % nt], 2), idx['down%d' % nt]])
        h = _gather_mean(ytab, comb, 2)
        if nt >= 2562:
            h = _conv_pair_big(h, idx['no%d' % lvl], params, 'u%dc1' % i,
                               'u%dc2' % i, oc, ocp, oc, ocp, i == 5)
        else:
            h = _conv_layer(h, idx['no%d' % lvl], params, 'u%dc1' % i,
                            oc, ocp, oc, ocp)
            h = _conv_layer(h, idx['no%d' % lvl], params, 'u%dc2' % i,
                            oc, ocp, oc, ocp, sigmoid=(i == 5))
    return h[:_NS[0], :3]
